# baseline (speedup 1.0000x reference)
import jax
import jax.numpy as jnp
from jax import lax
from jax.experimental import pallas as pl
from jax.experimental.pallas import tpu as pltpu
from jax.experimental.pallas import tpu_sc as plsc

B, L = 4096, 200
D, H, SP = 64, 128, 16
NRP = 8
NID = 100001
RBLK = 512
NPAD = 100352
NW = 32
BPW = B // NW
CH = 16
NCH = BPW // CH
S1, S2 = 104, 96
INV_L = 1.0 / L
BB3 = 512


def _proj_body(id_ref, wbot_ref, r_ref, wtop_ref, b_ref, idp_ref, rp_ref):
    idp_ref[...] = (jnp.dot(id_ref[...], wbot_ref[...],
                            preferred_element_type=jnp.float32)
                    * INV_L).astype(jnp.bfloat16)

    @pl.when(pl.program_id(0) == 0)
    def _():
        rp_ref[...] = ((jnp.dot(r_ref[...], wtop_ref[...],
                                preferred_element_type=jnp.float32)
                        + b_ref[...]) * INV_L).astype(jnp.bfloat16)


def _project(id_emd, ratings_emd, W, b):
    idp = jnp.pad(id_emd, ((0, NPAD - NID), (0, 0)))
    rp = jnp.pad(ratings_emd, ((0, NRP - ratings_emd.shape[0]), (0, 0)))
    return pl.pallas_call(
        _proj_body,
        grid=(NPAD // RBLK,),
        in_specs=[
            pl.BlockSpec((RBLK, D), lambda i: (i, 0)),
            pl.BlockSpec((D, H), lambda i: (0, 0)),
            pl.BlockSpec((NRP, D), lambda i: (0, 0)),
            pl.BlockSpec((D, H), lambda i: (0, 0)),
            pl.BlockSpec((1, H), lambda i: (0, 0)),
        ],
        out_specs=(
            pl.BlockSpec((RBLK, H), lambda i: (i, 0)),
            pl.BlockSpec((NRP, H), lambda i: (0, 0)),
        ),
        out_shape=(
            jax.ShapeDtypeStruct((NPAD, H), jnp.bfloat16),
            jax.ShapeDtypeStruct((NRP, H), jnp.bfloat16),
        ),
    )(idp, W[D:], rp, W[:D], b.reshape(1, H))


def _sc_body(uproj, iproj, urp, irp,
             i_id_seq, u_id_seq, u_rat_seq, i_rat_seq,
             gender, age, occ, zipc, genre,
             gtab, atab, otab, ztab, getab,
             ue_out, ie_out, dm_out,
             idall, rall, rows0, rows1, obuf, rpf, didx, dbuf, roffs,
             sem0, sem1):
    wid = lax.axis_index("c") * 16 + lax.axis_index("s")
    base = wid * BPW

    for t, (idx_hbm, tab_hbm) in enumerate(
            zip((gender, age, occ, zipc, genre),
                (gtab, atab, otab, ztab, getab))):
        pltpu.sync_copy(idx_hbm.at[pl.ds(base, BPW)], didx)
        pltpu.async_copy(tab_hbm.at[didx], dbuf, sem0).wait()
        pltpu.sync_copy(dbuf, dm_out.at[t, pl.ds(base, BPW)])

    def do_side(proj_hbm, rp_hbm, idseq_hbm, rseq_hbm, out_hbm):
        pltpu.sync_copy(rp_hbm, rpf)
        pltpu.sync_copy(idseq_hbm.at[pl.ds(base * L, BPW * L)], idall)
        pltpu.sync_copy(rseq_hbm.at[pl.ds(base * L, BPW * L)],
                        rall.at[pl.ds(0, BPW * L)])

        def fire(b, rows_ref, sem):
            pltpu.async_copy(proj_hbm.at[idall.at[pl.ds(b * L, S1)]],
                             rows_ref.at[pl.ds(0, S1)], sem)
            pltpu.async_copy(proj_hbm.at[idall.at[pl.ds(b * L + S1, S2)]],
                             rows_ref.at[pl.ds(S1, S2)], sem)

        def wait_rows(rows_ref, sem):
            pltpu.make_async_copy(proj_hbm.at[pl.ds(0, L)], rows_ref,
                                  sem).wait()

        def compute(b, rows_ref):
            def ga_body(t, _):
                rvec = rall[pl.ds(b * L + t * 16, 16)]
                offv = rvec * (H // 2)
                for k in range(16):
                    roffs[t * 16 + k] = offv[k]
                return 0

            lax.fori_loop(0, (L + 15) // 16, ga_body, 0)

            def l_body(l, accs):
                off = roffs[l]
                accs = list(accs)
                for j in range(4):
                    idv = plsc.bitcast(rows_ref[l, pl.ds(j * 16, 16)],
                                       jnp.bfloat16)
                    rv = plsc.bitcast(rpf[pl.ds(off + j * 16, 16)],
                                      jnp.bfloat16)
                    s = jnp.maximum(idv + rv, jnp.bfloat16(0.0))
                    ev, od = plsc.unpack(s, format=plsc.PackFormat.INTERLEAVED,
                                         preferred_element_type=jnp.float32)
                    accs[2 * j] = accs[2 * j] + ev
                    accs[2 * j + 1] = accs[2 * j + 1] + od
                return tuple(accs)

            accs = lax.fori_loop(
                0, L, l_body,
                tuple(jnp.zeros((16,), jnp.float32) for _ in range(8)))
            bo = lax.rem(b, CH)
            for j in range(8):
                obuf[pl.ds(bo * H + j * 16, 16)] = accs[j]

        fire(0, rows0, sem0)

        def pair_body(g, _):
            b0 = 2 * g
            b1 = 2 * g + 1
            fire(b1, rows1, sem1)
            wait_rows(rows0, sem0)
            compute(b0, rows0)

            @pl.when(g < BPW // 2 - 1)
            def _():
                fire(b0 + 2, rows0, sem0)

            wait_rows(rows1, sem1)
            compute(b1, rows1)

            @pl.when(lax.rem(b1, CH) == CH - 1)
            def _():
                pltpu.sync_copy(
                    obuf,
                    out_hbm.at[pl.ds((base + b1 - (CH - 1)) * H, CH * H)])
            return 0

        lax.fori_loop(0, BPW // 2, pair_body, 0)

    do_side(uproj, urp, i_id_seq, u_rat_seq, ue_out)
    do_side(iproj, irp, u_id_seq, i_rat_seq, ie_out)


def _sc_gather_reduce(uproj, urp, iproj, irp,
                      i_id_seq, u_id_seq, u_rat_seq, i_rat_seq,
                      gender, age, occ, zipc, genre,
                      gtab, atab, otab, ztab, getab):
    mesh = plsc.VectorSubcoreMesh(core_axis_name="c", subcore_axis_name="s")
    f = pl.kernel(
        _sc_body,
        out_type=(
            jax.ShapeDtypeStruct((B * H,), jnp.float32),
            jax.ShapeDtypeStruct((B * H,), jnp.float32),
            jax.ShapeDtypeStruct((5, B, SP), jnp.float32),
        ),
        mesh=mesh,
        compiler_params=pltpu.CompilerParams(use_tc_tiling_on_sc=False,
                                             needs_layout_passes=False),
        scratch_types=[
            pltpu.VMEM((BPW * L,), jnp.int32),
            pltpu.VMEM((BPW * L + 16,), jnp.int32),
            pltpu.VMEM((L, H // 2), jnp.int32),
            pltpu.VMEM((L, H // 2), jnp.int32),
            pltpu.VMEM((CH * H,), jnp.float32),
            pltpu.VMEM((NRP * H // 2,), jnp.int32),
            pltpu.VMEM((BPW,), jnp.int32),
            pltpu.VMEM((BPW, SP), jnp.float32),
            pltpu.SMEM((16 * ((L + 15) // 16),), jnp.int32),
            pltpu.SemaphoreType.DMA,
            pltpu.SemaphoreType.DMA,
        ],
    )
    uproj = lax.bitcast_convert_type(
        uproj.reshape(NPAD, H // 2, 2), jnp.int32)
    iproj = lax.bitcast_convert_type(
        iproj.reshape(NPAD, H // 2, 2), jnp.int32)
    urp = lax.bitcast_convert_type(
        urp.reshape(NRP * H // 2, 2), jnp.int32)
    irp = lax.bitcast_convert_type(
        irp.reshape(NRP * H // 2, 2), jnp.int32)
    ue, ie, dm = f(uproj, iproj, urp, irp,
                   i_id_seq.reshape(-1), u_id_seq.reshape(-1),
                   u_rat_seq.reshape(-1), i_rat_seq.reshape(-1),
                   gender, age, occ, zipc, genre,
                   gtab, atab, otab, ztab, getab)
    return ue.reshape(B, H), ie.reshape(B, H), dm


def _head_body(ue_ref, ie_ref, dm_ref, lab_ref, wu_ref, wi_ref, wd_ref,
               b_ref, w2_ref, b2_ref, out_ref):
    i = pl.program_id(0)
    x = jnp.dot(ue_ref[...], wu_ref[...], preferred_element_type=jnp.float32)
    x = x + jnp.dot(ie_ref[...], wi_ref[...],
                    preferred_element_type=jnp.float32)
    for t in range(5):
        x = x + jnp.dot(dm_ref[t], wd_ref[t],
                        preferred_element_type=jnp.float32)
    h = jnp.maximum(x + b_ref[...], 0.0)
    logits = jnp.sum(h * w2_ref[...], axis=1) + b2_ref[0, 0]
    y = lab_ref[...].astype(jnp.float32)
    terms = (jnp.maximum(logits, 0.0) - logits * y
             + jnp.log1p(jnp.exp(-jnp.abs(logits))))
    part = jnp.sum(terms).reshape(1, 1) * (1.0 / B)
    prev = jnp.where(i == 0, jnp.zeros((1, 1), jnp.float32), out_ref[...])
    out_ref[...] = prev + part


_PERM = [32 * j + 2 * w + hh for j in range(4) for hh in (0, 1)
         for w in range(16)]


def _head(ue, ie, dm, labels, fc1_W, fc1_b, fc2_W, fc2_b):
    perm = jnp.asarray(_PERM, dtype=jnp.int32)
    wu = fc1_W[:H][perm]
    wi = fc1_W[H:2 * H][perm]
    wd = fc1_W[2 * H:].reshape(5, SP, H)
    out = pl.pallas_call(
        _head_body,
        grid=(B // BB3,),
        in_specs=[
            pl.BlockSpec((BB3, H), lambda i: (i, 0)),
            pl.BlockSpec((BB3, H), lambda i: (i, 0)),
            pl.BlockSpec((5, BB3, SP), lambda i: (0, i, 0)),
            pl.BlockSpec((BB3,), lambda i: (i,)),
            pl.BlockSpec((H, H), lambda i: (0, 0)),
            pl.BlockSpec((H, H), lambda i: (0, 0)),
            pl.BlockSpec((5, SP, H), lambda i: (0, 0, 0)),
            pl.BlockSpec((1, H), lambda i: (0, 0)),
            pl.BlockSpec((1, H), lambda i: (0, 0)),
            pl.BlockSpec((1, 1), lambda i: (0, 0)),
        ],
        out_specs=pl.BlockSpec((1, 1), lambda i: (0, 0)),
        out_shape=jax.ShapeDtypeStruct((1, 1), jnp.float32),
    )(ue, ie, dm, labels, wu, wi, wd, fc1_b.reshape(1, H),
      fc2_W.reshape(1, H), fc2_b.reshape(1, 1))
    return out[0, 0]


def kernel(u_rating_seq, i_id_seq, i_rating_seq, u_id_seq, gender, age,
           occupation, zip_code, genre, labels, u_ratings_emd, u_id_emd,
           u_proj_W, u_proj_b, i_ratings_emd, i_id_emd, i_proj_W, i_proj_b,
           gender_tab, age_tab, occ_tab, zip_tab, genre_tab, fc1_W, fc1_b,
           fc2_W, fc2_b):
    uproj, urp = _project(u_id_emd, u_ratings_emd, u_proj_W, u_proj_b)
    iproj, irp = _project(i_id_emd, i_ratings_emd, i_proj_W, i_proj_b)
    ue, ie, dm = _sc_gather_reduce(
        uproj, urp, iproj, irp,
        i_id_seq.astype(jnp.int32), u_id_seq.astype(jnp.int32),
        u_rating_seq.astype(jnp.int32), i_rating_seq.astype(jnp.int32),
        gender.astype(jnp.int32), age.astype(jnp.int32),
        occupation.astype(jnp.int32), zip_code.astype(jnp.int32),
        genre.astype(jnp.int32),
        gender_tab, age_tab, occ_tab, zip_tab, genre_tab)
    return _head(ue, ie, dm, labels.astype(jnp.int32),
                 fc1_W, fc1_b, fc2_W, fc2_b)


# speedup vs baseline: 1.5656x; 1.5656x over previous
import jax
import jax.numpy as jnp
from jax import lax
from jax.experimental import pallas as pl
from jax.experimental.pallas import tpu as pltpu
from jax.experimental.pallas import tpu_sc as plsc

B, L = 4096, 200
D, H, SP = 64, 128, 16
NRP = 8
NID = 100001
RBLK = 512
NPAD = 100352
NW = 32
BPW = B // NW
CH = 16
NCH = BPW // CH
S1, S2 = 104, 96
INV_L = 1.0 / L
BB3 = 512


def _pack_halves(y):
    lo = lax.bitcast_convert_type(y[:, :64].astype(jnp.bfloat16),
                                  jnp.uint16).astype(jnp.uint32)
    hi = lax.bitcast_convert_type(y[:, 64:].astype(jnp.bfloat16),
                                  jnp.uint16).astype(jnp.uint32)
    return lax.bitcast_convert_type(lo | (hi << 16), jnp.int32)


def _proj_body(id_ref, wbot_ref, r_ref, wtop_ref, b_ref, idp_ref, rp_ref):
    idp_ref[...] = _pack_halves(jnp.dot(id_ref[...], wbot_ref[...],
                                        preferred_element_type=jnp.float32)
                                * INV_L)

    @pl.when(pl.program_id(0) == 0)
    def _():
        rp_ref[...] = _pack_halves((jnp.dot(r_ref[...], wtop_ref[...],
                                            preferred_element_type=jnp.float32)
                                    + b_ref[...]) * INV_L)


def _project(id_emd, ratings_emd, W, b):
    rp = jnp.pad(ratings_emd, ((0, NRP - ratings_emd.shape[0]), (0, 0)))
    return pl.pallas_call(
        _proj_body,
        grid=(NPAD // RBLK,),
        in_specs=[
            pl.BlockSpec((RBLK, D), lambda i: (i, 0)),
            pl.BlockSpec((D, H), lambda i: (0, 0)),
            pl.BlockSpec((NRP, D), lambda i: (0, 0)),
            pl.BlockSpec((D, H), lambda i: (0, 0)),
            pl.BlockSpec((1, H), lambda i: (0, 0)),
        ],
        out_specs=(
            pl.BlockSpec((RBLK, H // 2), lambda i: (i, 0)),
            pl.BlockSpec((NRP, H // 2), lambda i: (0, 0)),
        ),
        out_shape=(
            jax.ShapeDtypeStruct((NPAD, H // 2), jnp.int32),
            jax.ShapeDtypeStruct((NRP, H // 2), jnp.int32),
        ),
    )(id_emd, W[D:], rp, W[:D], b.reshape(1, H))


def _sc_body(uproj, iproj, urp, irp,
             i_id_seq, u_id_seq, u_rat_seq, i_rat_seq,
             gender, age, occ, zipc, genre,
             gtab, atab, otab, ztab, getab,
             ue_out, ie_out, dm_out,
             idall, rall, rows0, rows1, obuf, rpf, didx, dbuf, roffs,
             sem0, sem1):
    wid = lax.axis_index("c") * 16 + lax.axis_index("s")
    base = wid * BPW

    for t, (idx_hbm, tab_hbm) in enumerate(
            zip((gender, age, occ, zipc, genre),
                (gtab, atab, otab, ztab, getab))):
        pltpu.sync_copy(idx_hbm.at[pl.ds(base, BPW)], didx)
        pltpu.async_copy(tab_hbm.at[didx], dbuf, sem0).wait()
        pltpu.sync_copy(dbuf, dm_out.at[t, pl.ds(base, BPW)])

    def do_side(proj_hbm, rp_hbm, idseq_hbm, rseq_hbm, out_hbm):
        pltpu.sync_copy(rp_hbm, rpf)
        pltpu.sync_copy(idseq_hbm.at[pl.ds(base, BPW)], idall)
        pltpu.sync_copy(rseq_hbm.at[pl.ds(base, BPW)],
                        rall.at[pl.ds(0, BPW)])

        def fire(b, rows_ref, sem):
            pltpu.async_copy(proj_hbm.at[idall.at[b, pl.ds(0, S1)]],
                             rows_ref.at[pl.ds(0, S1)], sem)
            pltpu.async_copy(proj_hbm.at[idall.at[b, pl.ds(S1, S2)]],
                             rows_ref.at[pl.ds(S1, S2)], sem)

        def wait_rows(rows_ref, sem):
            pltpu.make_async_copy(proj_hbm.at[pl.ds(0, L)], rows_ref,
                                  sem).wait()

        def compute(b, rows_ref):
            def ga_body(t, _):
                rvec = rall[b, pl.ds(t * 16, 16)]
                offv = rvec * (H // 2)
                for k in range(16):
                    roffs[t * 16 + k] = offv[k]
                return 0

            lax.fori_loop(0, (L + 15) // 16, ga_body, 0)

            def l_body(l, accs):
                off = roffs[l]
                accs = list(accs)
                for j in range(4):
                    idv = plsc.bitcast(rows_ref[l, pl.ds(j * 16, 16)],
                                       jnp.bfloat16)
                    rv = plsc.bitcast(rpf[pl.ds(off + j * 16, 16)],
                                      jnp.bfloat16)
                    s = jnp.maximum(idv + rv, jnp.bfloat16(0.0))
                    ev, od = plsc.unpack(s, format=plsc.PackFormat.INTERLEAVED,
                                         preferred_element_type=jnp.float32)
                    accs[2 * j] = accs[2 * j] + ev
                    accs[2 * j + 1] = accs[2 * j + 1] + od
                return tuple(accs)

            accs = lax.fori_loop(
                0, L, l_body,
                tuple(jnp.zeros((16,), jnp.float32) for _ in range(8)))
            bo = lax.rem(b, CH)
            for j in range(8):
                obuf[pl.ds(bo * H + j * 16, 16)] = accs[j]

        fire(0, rows0, sem0)

        def pair_body(g, _):
            b0 = 2 * g
            b1 = 2 * g + 1
            fire(b1, rows1, sem1)
            wait_rows(rows0, sem0)
            compute(b0, rows0)

            @pl.when(g < BPW // 2 - 1)
            def _():
                fire(b0 + 2, rows0, sem0)

            wait_rows(rows1, sem1)
            compute(b1, rows1)

            @pl.when(lax.rem(b1, CH) == CH - 1)
            def _():
                pltpu.sync_copy(
                    obuf,
                    out_hbm.at[pl.ds((base + b1 - (CH - 1)) * H, CH * H)])
            return 0

        lax.fori_loop(0, BPW // 2, pair_body, 0)

    do_side(uproj, urp, i_id_seq, u_rat_seq, ue_out)
    do_side(iproj, irp, u_id_seq, i_rat_seq, ie_out)


def _sc_gather_reduce(uproj, urp, iproj, irp,
                      i_id_seq, u_id_seq, u_rat_seq, i_rat_seq,
                      gender, age, occ, zipc, genre,
                      gtab, atab, otab, ztab, getab):
    mesh = plsc.VectorSubcoreMesh(core_axis_name="c", subcore_axis_name="s")
    f = pl.kernel(
        _sc_body,
        out_type=(
            jax.ShapeDtypeStruct((B * H,), jnp.float32),
            jax.ShapeDtypeStruct((B * H,), jnp.float32),
            jax.ShapeDtypeStruct((5, B, SP), jnp.float32),
        ),
        mesh=mesh,
        compiler_params=pltpu.CompilerParams(use_tc_tiling_on_sc=False,
                                             needs_layout_passes=False),
        scratch_types=[
            pltpu.VMEM((BPW, L), jnp.int32),
            pltpu.VMEM((BPW + 1, L), jnp.int32),
            pltpu.VMEM((L, H // 2), jnp.int32),
            pltpu.VMEM((L, H // 2), jnp.int32),
            pltpu.VMEM((CH * H,), jnp.float32),
            pltpu.VMEM((NRP * H // 2,), jnp.int32),
            pltpu.VMEM((BPW,), jnp.int32),
            pltpu.VMEM((BPW, SP), jnp.float32),
            pltpu.SMEM((16 * ((L + 15) // 16),), jnp.int32),
            pltpu.SemaphoreType.DMA,
            pltpu.SemaphoreType.DMA,
        ],
    )
    ue, ie, dm = f(uproj, iproj, urp.reshape(-1), irp.reshape(-1),
                   i_id_seq, u_id_seq,
                   u_rat_seq, i_rat_seq,
                   gender, age, occ, zipc, genre,
                   gtab, atab, otab, ztab, getab)
    return ue.reshape(B, H), ie.reshape(B, H), dm


def _head_body(ue_ref, ie_ref, dm_ref, lab_ref, wu_ref, wi_ref, wd_ref,
               b_ref, w2_ref, b2_ref, out_ref):
    i = pl.program_id(0)
    x = jnp.dot(ue_ref[...], wu_ref[...], preferred_element_type=jnp.float32)
    x = x + jnp.dot(ie_ref[...], wi_ref[...],
                    preferred_element_type=jnp.float32)
    for t in range(5):
        x = x + jnp.dot(dm_ref[t], wd_ref[t],
                        preferred_element_type=jnp.float32)
    h = jnp.maximum(x + b_ref[...], 0.0)
    logits = jnp.sum(h * w2_ref[...], axis=1) + b2_ref[0, 0]
    y = lab_ref[...].astype(jnp.float32)
    terms = (jnp.maximum(logits, 0.0) - logits * y
             + jnp.log1p(jnp.exp(-jnp.abs(logits))))
    part = jnp.sum(terms).reshape(1, 1) * (1.0 / B)
    prev = jnp.where(i == 0, jnp.zeros((1, 1), jnp.float32), out_ref[...])
    out_ref[...] = prev + part


_PERM = [(q // 2) * 16 + w + 64 * (q % 2) for q in range(8)
         for w in range(16)]


def _head(ue, ie, dm, labels, fc1_W, fc1_b, fc2_W, fc2_b):
    perm = jnp.asarray(_PERM, dtype=jnp.int32)
    wu = fc1_W[:H][perm]
    wi = fc1_W[H:2 * H][perm]
    wd = fc1_W[2 * H:].reshape(5, SP, H)
    out = pl.pallas_call(
        _head_body,
        grid=(B // BB3,),
        in_specs=[
            pl.BlockSpec((BB3, H), lambda i: (i, 0)),
            pl.BlockSpec((BB3, H), lambda i: (i, 0)),
            pl.BlockSpec((5, BB3, SP), lambda i: (0, i, 0)),
            pl.BlockSpec((BB3,), lambda i: (i,)),
            pl.BlockSpec((H, H), lambda i: (0, 0)),
            pl.BlockSpec((H, H), lambda i: (0, 0)),
            pl.BlockSpec((5, SP, H), lambda i: (0, 0, 0)),
            pl.BlockSpec((1, H), lambda i: (0, 0)),
            pl.BlockSpec((1, H), lambda i: (0, 0)),
            pl.BlockSpec((1, 1), lambda i: (0, 0)),
        ],
        out_specs=pl.BlockSpec((1, 1), lambda i: (0, 0)),
        out_shape=jax.ShapeDtypeStruct((1, 1), jnp.float32),
    )(ue, ie, dm, labels, wu, wi, wd, fc1_b.reshape(1, H),
      fc2_W.reshape(1, H), fc2_b.reshape(1, 1))
    return out[0, 0]


def kernel(u_rating_seq, i_id_seq, i_rating_seq, u_id_seq, gender, age,
           occupation, zip_code, genre, labels, u_ratings_emd, u_id_emd,
           u_proj_W, u_proj_b, i_ratings_emd, i_id_emd, i_proj_W, i_proj_b,
           gender_tab, age_tab, occ_tab, zip_tab, genre_tab, fc1_W, fc1_b,
           fc2_W, fc2_b):
    uproj, urp = _project(u_id_emd, u_ratings_emd, u_proj_W, u_proj_b)
    iproj, irp = _project(i_id_emd, i_ratings_emd, i_proj_W, i_proj_b)
    ue, ie, dm = _sc_gather_reduce(
        uproj, urp, iproj, irp,
        i_id_seq.astype(jnp.int32), u_id_seq.astype(jnp.int32),
        u_rating_seq.astype(jnp.int32), i_rating_seq.astype(jnp.int32),
        gender.astype(jnp.int32), age.astype(jnp.int32),
        occupation.astype(jnp.int32), zip_code.astype(jnp.int32),
        genre.astype(jnp.int32),
        gender_tab, age_tab, occ_tab, zip_tab, genre_tab)
    return _head(ue, ie, dm, labels.astype(jnp.int32),
                 fc1_W, fc1_b, fc2_W, fc2_b)


# speedup vs baseline: 2.4739x; 1.5802x over previous
import jax
import jax.numpy as jnp
from jax import lax
from jax.experimental import pallas as pl
from jax.experimental.pallas import tpu as pltpu
from jax.experimental.pallas import tpu_sc as plsc

B, L = 4096, 200
D, H, SP = 64, 128, 16
NRP = 8
NID = 100001
RBLK = 2048
NPAD = 100352
NW = 32
BPW = B // NW
CH = 16
NCH = BPW // CH
S1, S2 = 104, 96
INV_L = 1.0 / L
BB3 = 512


def _pack_halves(y):
    lo = lax.bitcast_convert_type(y[:, :64].astype(jnp.bfloat16),
                                  jnp.uint16).astype(jnp.uint32)
    hi = lax.bitcast_convert_type(y[:, 64:].astype(jnp.bfloat16),
                                  jnp.uint16).astype(jnp.uint32)
    return lax.bitcast_convert_type(lo | (hi << 16), jnp.int32)


def _proj_body(id_ref, wbot_ref, r_ref, wtop_ref, b_ref, idp_ref, rp_ref):
    idp_ref[...] = _pack_halves(jnp.dot(id_ref[...], wbot_ref[...],
                                        preferred_element_type=jnp.float32)
                                * INV_L)

    @pl.when(pl.program_id(0) == 0)
    def _():
        rp_ref[...] = _pack_halves((jnp.dot(r_ref[...], wtop_ref[...],
                                            preferred_element_type=jnp.float32)
                                    + b_ref[...]) * INV_L)


def _project(id_emd, ratings_emd, W, b):
    rp = jnp.pad(ratings_emd, ((0, NRP - ratings_emd.shape[0]), (0, 0)))
    return pl.pallas_call(
        _proj_body,
        grid=(NPAD // RBLK,),
        in_specs=[
            pl.BlockSpec((RBLK, D), lambda i: (i, 0)),
            pl.BlockSpec((D, H), lambda i: (0, 0)),
            pl.BlockSpec((NRP, D), lambda i: (0, 0)),
            pl.BlockSpec((D, H), lambda i: (0, 0)),
            pl.BlockSpec((1, H), lambda i: (0, 0)),
        ],
        out_specs=(
            pl.BlockSpec((RBLK, H // 2), lambda i: (i, 0)),
            pl.BlockSpec((NRP, H // 2), lambda i: (0, 0)),
        ),
        out_shape=(
            jax.ShapeDtypeStruct((NPAD, H // 2), jnp.int32),
            jax.ShapeDtypeStruct((NRP, H // 2), jnp.int32),
        ),
    )(id_emd, W[D:], rp, W[:D], b.reshape(1, H))


def _sc_demog_body(gender, age, occ, zipc, genre,
                   gtab, atab, otab, ztab, getab,
                   dm_out, didx, dbuf, sem0):
    wid = lax.axis_index("c") * 16 + lax.axis_index("s")
    base = wid * BPW
    for t, (idx_hbm, tab_hbm) in enumerate(
            zip((gender, age, occ, zipc, genre),
                (gtab, atab, otab, ztab, getab))):
        pltpu.sync_copy(idx_hbm.at[pl.ds(base, BPW)], didx)
        pltpu.async_copy(tab_hbm.at[didx], dbuf, sem0).wait()
        pltpu.sync_copy(dbuf, dm_out.at[t, pl.ds(base, BPW)])


def _sc_body(proj, rp, idseq, rseq,
             out_flat,
             idall, rall, rows0, rows1, obuf, rpf, didx, dbuf, roffs,
             sem0, sem1):
    wid = lax.axis_index("c") * 16 + lax.axis_index("s")
    base = wid * BPW

    def do_side(proj_hbm, rp_hbm, idseq_hbm, rseq_hbm, out_hbm):
        pltpu.sync_copy(rp_hbm, rpf)
        pltpu.sync_copy(idseq_hbm.at[pl.ds(base, BPW)], idall)
        pltpu.sync_copy(rseq_hbm.at[pl.ds(base, BPW)],
                        rall.at[pl.ds(0, BPW)])

        def fire(b, rows_ref, sem):
            pltpu.async_copy(proj_hbm.at[idall.at[b, pl.ds(0, S1)]],
                             rows_ref.at[pl.ds(0, S1)], sem)
            pltpu.async_copy(proj_hbm.at[idall.at[b, pl.ds(S1, S2)]],
                             rows_ref.at[pl.ds(S1, S2)], sem)

        def wait_rows(rows_ref, sem):
            pltpu.make_async_copy(proj_hbm.at[pl.ds(0, L)], rows_ref,
                                  sem).wait()

        def compute(b, rows_ref):
            def ga_body(t, _):
                rvec = rall[b, pl.ds(t * 16, 16)]
                offv = rvec * (H // 2)
                for k in range(16):
                    roffs[t * 16 + k] = offv[k]
                return 0

            lax.fori_loop(0, (L + 15) // 16, ga_body, 0)

            def l_body(l, accs):
                off = roffs[l]
                accs = list(accs)
                for j in range(4):
                    idv = plsc.bitcast(rows_ref[l, pl.ds(j * 16, 16)],
                                       jnp.bfloat16)
                    rv = plsc.bitcast(rpf[pl.ds(off + j * 16, 16)],
                                      jnp.bfloat16)
                    s = jnp.maximum(idv + rv, jnp.bfloat16(0.0))
                    ev, od = plsc.unpack(s, format=plsc.PackFormat.INTERLEAVED,
                                         preferred_element_type=jnp.float32)
                    accs[2 * j] = accs[2 * j] + ev
                    accs[2 * j + 1] = accs[2 * j + 1] + od
                return tuple(accs)

            accs = lax.fori_loop(
                0, L, l_body,
                tuple(jnp.zeros((16,), jnp.float32) for _ in range(8)))
            bo = lax.rem(b, CH)
            for j in range(8):
                obuf[pl.ds(bo * H + j * 16, 16)] = accs[j]

        fire(0, rows0, sem0)

        def pair_body(g, _):
            b0 = 2 * g
            b1 = 2 * g + 1
            fire(b1, rows1, sem1)
            wait_rows(rows0, sem0)
            compute(b0, rows0)

            @pl.when(g < BPW // 2 - 1)
            def _():
                fire(b0 + 2, rows0, sem0)

            wait_rows(rows1, sem1)
            compute(b1, rows1)

            @pl.when(lax.rem(b1, CH) == CH - 1)
            def _():
                pltpu.sync_copy(
                    obuf,
                    out_hbm.at[pl.ds((base + b1 - (CH - 1)) * H, CH * H)])
            return 0

        lax.fori_loop(0, BPW // 2, pair_body, 0)

    do_side(proj, rp, idseq, rseq, out_flat)


_SC_MESH = plsc.VectorSubcoreMesh(core_axis_name="c", subcore_axis_name="s")
_SC_PARAMS = pltpu.CompilerParams(use_tc_tiling_on_sc=False,
                                  needs_layout_passes=False)


def _sc_side(proj, rp, idseq, rseq):
    f = pl.kernel(
        _sc_body,
        out_type=jax.ShapeDtypeStruct((B * H,), jnp.float32),
        mesh=_SC_MESH,
        compiler_params=_SC_PARAMS,
        scratch_types=[
            pltpu.VMEM((BPW, L), jnp.int32),
            pltpu.VMEM((BPW + 1, L), jnp.int32),
            pltpu.VMEM((L, H // 2), jnp.int32),
            pltpu.VMEM((L, H // 2), jnp.int32),
            pltpu.VMEM((CH * H,), jnp.float32),
            pltpu.VMEM((NRP * H // 2,), jnp.int32),
            pltpu.VMEM((BPW,), jnp.int32),
            pltpu.VMEM((BPW, SP), jnp.float32),
            pltpu.SMEM((16 * ((L + 15) // 16),), jnp.int32),
            pltpu.SemaphoreType.DMA,
            pltpu.SemaphoreType.DMA,
        ],
    )
    return f(proj, rp.reshape(-1), idseq, rseq).reshape(B, H)


def _sc_demog(gender, age, occ, zipc, genre,
              gtab, atab, otab, ztab, getab):
    f = pl.kernel(
        _sc_demog_body,
        out_type=jax.ShapeDtypeStruct((5, B, SP), jnp.float32),
        mesh=_SC_MESH,
        compiler_params=_SC_PARAMS,
        scratch_types=[
            pltpu.VMEM((BPW,), jnp.int32),
            pltpu.VMEM((BPW, SP), jnp.float32),
            pltpu.SemaphoreType.DMA,
        ],
    )
    return f(gender, age, occ, zipc, genre,
             gtab, atab, otab, ztab, getab)


def _head_body(ue_ref, ie_ref, dm_ref, lab_ref, wu_ref, wi_ref, wd_ref,
               b_ref, w2_ref, b2_ref, out_ref):
    i = pl.program_id(0)
    x = jnp.dot(ue_ref[...], wu_ref[...], preferred_element_type=jnp.float32)
    x = x + jnp.dot(ie_ref[...], wi_ref[...],
                    preferred_element_type=jnp.float32)
    for t in range(5):
        x = x + jnp.dot(dm_ref[t], wd_ref[t],
                        preferred_element_type=jnp.float32)
    h = jnp.maximum(x + b_ref[...], 0.0)
    logits = jnp.sum(h * w2_ref[...], axis=1) + b2_ref[0, 0]
    y = lab_ref[...].astype(jnp.float32)
    terms = (jnp.maximum(logits, 0.0) - logits * y
             + jnp.log1p(jnp.exp(-jnp.abs(logits))))
    part = jnp.sum(terms).reshape(1, 1) * (1.0 / B)
    prev = jnp.where(i == 0, jnp.zeros((1, 1), jnp.float32), out_ref[...])
    out_ref[...] = prev + part


_PERM = [(q // 2) * 16 + w + 64 * (q % 2) for q in range(8)
         for w in range(16)]


def _head(ue, ie, dm, labels, fc1_W, fc1_b, fc2_W, fc2_b):
    perm = jnp.asarray(_PERM, dtype=jnp.int32)
    wu = fc1_W[:H][perm]
    wi = fc1_W[H:2 * H][perm]
    wd = fc1_W[2 * H:].reshape(5, SP, H)
    out = pl.pallas_call(
        _head_body,
        grid=(B // BB3,),
        in_specs=[
            pl.BlockSpec((BB3, H), lambda i: (i, 0)),
            pl.BlockSpec((BB3, H), lambda i: (i, 0)),
            pl.BlockSpec((5, BB3, SP), lambda i: (0, i, 0)),
            pl.BlockSpec((BB3,), lambda i: (i,)),
            pl.BlockSpec((H, H), lambda i: (0, 0)),
            pl.BlockSpec((H, H), lambda i: (0, 0)),
            pl.BlockSpec((5, SP, H), lambda i: (0, 0, 0)),
            pl.BlockSpec((1, H), lambda i: (0, 0)),
            pl.BlockSpec((1, H), lambda i: (0, 0)),
            pl.BlockSpec((1, 1), lambda i: (0, 0)),
        ],
        out_specs=pl.BlockSpec((1, 1), lambda i: (0, 0)),
        out_shape=jax.ShapeDtypeStruct((1, 1), jnp.float32),
    )(ue, ie, dm, labels, wu, wi, wd, fc1_b.reshape(1, H),
      fc2_W.reshape(1, H), fc2_b.reshape(1, 1))
    return out[0, 0]


def kernel(u_rating_seq, i_id_seq, i_rating_seq, u_id_seq, gender, age,
           occupation, zip_code, genre, labels, u_ratings_emd, u_id_emd,
           u_proj_W, u_proj_b, i_ratings_emd, i_id_emd, i_proj_W, i_proj_b,
           gender_tab, age_tab, occ_tab, zip_tab, genre_tab, fc1_W, fc1_b,
           fc2_W, fc2_b):
    uproj, urp = _project(u_id_emd, u_ratings_emd, u_proj_W, u_proj_b)
    ue = _sc_side(uproj, urp, i_id_seq.astype(jnp.int32),
                  u_rating_seq.astype(jnp.int32))
    iproj, irp = _project(i_id_emd, i_ratings_emd, i_proj_W, i_proj_b)
    ie = _sc_side(iproj, irp, u_id_seq.astype(jnp.int32),
                  i_rating_seq.astype(jnp.int32))
    dm = _sc_demog(
        gender.astype(jnp.int32), age.astype(jnp.int32),
        occupation.astype(jnp.int32), zip_code.astype(jnp.int32),
        genre.astype(jnp.int32),
        gender_tab, age_tab, occ_tab, zip_tab, genre_tab)
    return _head(ue, ie, dm, labels.astype(jnp.int32),
                 fc1_W, fc1_b, fc2_W, fc2_b)
